# baseline (speedup 1.0000x reference)
import jax
import jax.numpy as jnp
from jax.experimental import pallas as pl
from jax.experimental.pallas import tpu as pltpu

_BM = 512
_KC = 1024


def _mlp_kernel(coef_ref, x_ref, we_ref, wo_ref, b1_ref, w2_ref, b2_ref,
                out_ref):
    T = x_ref.shape[1]
    c = coef_ref[...].astype(jnp.bfloat16)
    h = None
    for c0 in range(0, T, _KC):
        xb = x_ref[:, c0:c0 + _KC]
        v0 = (xb & 1).astype(jnp.bfloat16)
        v1 = (xb >> 1).astype(jnp.bfloat16)
        p = v0 * v1
        g0 = c[0:1, 0:1] + c[0:1, 1:2] * v0 + c[0:1, 2:3] * v1 + c[0:1, 3:4] * p
        g1 = c[0:1, 4:5] + c[0:1, 5:6] * v0 + c[0:1, 6:7] * v1 + c[0:1, 7:8] * p
        d = jnp.dot(g0, we_ref[c0:c0 + _KC, :],
                    preferred_element_type=jnp.float32)
        d = d + jnp.dot(g1, wo_ref[c0:c0 + _KC, :],
                        preferred_element_type=jnp.float32)
        h = d if h is None else h + d
    h = jnp.maximum(h + b1_ref[...], 0.0)
    out_ref[...] = jnp.dot(h, w2_ref[...],
                           preferred_element_type=jnp.float32) + b2_ref[...]


def kernel(x, table, W1, b1, W2, b2):
    B, T = x.shape
    d_hid = W1.shape[1]
    d_out = W2.shape[1]
    t = table
    coef = jnp.stack([
        t[0, 0], t[1, 0] - t[0, 0], t[2, 0] - t[0, 0],
        t[3, 0] - t[2, 0] - t[1, 0] + t[0, 0],
        t[0, 1], t[1, 1] - t[0, 1], t[2, 1] - t[0, 1],
        t[3, 1] - t[2, 1] - t[1, 1] + t[0, 1],
    ]).reshape(1, 8)
    b1r = b1.reshape(1, d_hid)
    b2r = b2.reshape(1, d_out)
    we = W1[0::2, :].astype(jnp.bfloat16)
    wo = W1[1::2, :].astype(jnp.bfloat16)
    return pl.pallas_call(
        _mlp_kernel,
        grid=(B // _BM,),
        in_specs=[
            pl.BlockSpec((1, 8), lambda i: (0, 0)),
            pl.BlockSpec((_BM, T), lambda i: (i, 0)),
            pl.BlockSpec((T, d_hid), lambda i: (0, 0)),
            pl.BlockSpec((T, d_hid), lambda i: (0, 0)),
            pl.BlockSpec((1, d_hid), lambda i: (0, 0)),
            pl.BlockSpec((d_hid, d_out), lambda i: (0, 0)),
            pl.BlockSpec((1, d_out), lambda i: (0, 0)),
        ],
        out_specs=pl.BlockSpec((_BM, d_out), lambda i: (i, 0)),
        out_shape=jax.ShapeDtypeStruct((B, d_out), jnp.float32),
        compiler_params=pltpu.CompilerParams(
            dimension_semantics=("arbitrary",)),
    )(coef, x, we, wo, b1r, W2, b2r)


# speedup vs baseline: 1.3799x; 1.3799x over previous
import jax
import jax.numpy as jnp
from jax.experimental import pallas as pl
from jax.experimental.pallas import tpu as pltpu

_BM = 512
_KC = 1024


def _mlp_kernel(coef_ref, x_ref, w1_ref, b1_ref, w2_ref, b2_ref, out_ref,
                we_ref, wo_ref):
    T = x_ref.shape[1]

    @pl.when(pl.program_id(0) == 0)
    def _split():
        w1b = w1_ref[...].astype(jnp.bfloat16)
        wpk = pltpu.bitcast(w1b, jnp.int32)
        we_ref[...] = pltpu.bitcast(wpk << 16, jnp.float32).astype(jnp.bfloat16)
        wo_ref[...] = pltpu.bitcast(
            wpk & jnp.int32(-65536), jnp.float32).astype(jnp.bfloat16)

    c = coef_ref[...].astype(jnp.bfloat16)
    h = None
    for c0 in range(0, T, _KC):
        xb = x_ref[:, c0:c0 + _KC]
        v0 = (xb & 1).astype(jnp.bfloat16)
        v1 = (xb >> 1).astype(jnp.bfloat16)
        p = v0 * v1
        g0 = c[0:1, 0:1] + c[0:1, 1:2] * v0 + c[0:1, 2:3] * v1 + c[0:1, 3:4] * p
        g1 = c[0:1, 4:5] + c[0:1, 5:6] * v0 + c[0:1, 6:7] * v1 + c[0:1, 7:8] * p
        d = jnp.dot(g0, we_ref[c0:c0 + _KC, :],
                    preferred_element_type=jnp.float32)
        d = d + jnp.dot(g1, wo_ref[c0:c0 + _KC, :],
                        preferred_element_type=jnp.float32)
        h = d if h is None else h + d
    h = jnp.maximum(h + b1_ref[...], 0.0)
    out_ref[...] = jnp.dot(h, w2_ref[...],
                           preferred_element_type=jnp.float32) + b2_ref[...]


def kernel(x, table, W1, b1, W2, b2):
    B, T = x.shape
    d_in, d_hid = W1.shape
    d_out = W2.shape[1]
    t = table
    coef = jnp.stack([
        t[0, 0], t[1, 0] - t[0, 0], t[2, 0] - t[0, 0],
        t[3, 0] - t[2, 0] - t[1, 0] + t[0, 0],
        t[0, 1], t[1, 1] - t[0, 1], t[2, 1] - t[0, 1],
        t[3, 1] - t[2, 1] - t[1, 1] + t[0, 1],
    ]).reshape(1, 8)
    b1r = b1.reshape(1, d_hid)
    b2r = b2.reshape(1, d_out)
    return pl.pallas_call(
        _mlp_kernel,
        grid=(B // _BM,),
        in_specs=[
            pl.BlockSpec((1, 8), lambda i: (0, 0)),
            pl.BlockSpec((_BM, T), lambda i: (i, 0)),
            pl.BlockSpec((d_in, d_hid), lambda i: (0, 0)),
            pl.BlockSpec((1, d_hid), lambda i: (0, 0)),
            pl.BlockSpec((d_hid, d_out), lambda i: (0, 0)),
            pl.BlockSpec((1, d_out), lambda i: (0, 0)),
        ],
        out_specs=pl.BlockSpec((_BM, d_out), lambda i: (i, 0)),
        out_shape=jax.ShapeDtypeStruct((B, d_out), jnp.float32),
        scratch_shapes=[
            pltpu.VMEM((T, d_hid), jnp.bfloat16),
            pltpu.VMEM((T, d_hid), jnp.bfloat16),
        ],
        compiler_params=pltpu.CompilerParams(
            dimension_semantics=("arbitrary",)),
    )(coef, x, W1, b1r, W2, b2r)
